# baseline (speedup 1.0000x reference)
import functools

import jax
import jax.numpy as jnp
from jax import lax
from jax.experimental import pallas as pl
from jax.experimental.pallas import tpu as pltpu
from jax.experimental.pallas import tpu_sc as plsc

S = 64.0
B = 1024
C = 100000

_BLK_R = 32
_RB = B // _BLK_R
_G = _BLK_R // 8


def _tc_stream_body(lab_sref, *refs):
    cos_ref = refs[0]
    tile_refs = refs[1:1 + _BLK_R]
    lab_ref = refs[1 + _BLK_R]
    sum_ref = refs[2 + _BLK_R]
    stage_ref = refs[3 + _BLK_R]

    x = cos_ref[...] * S
    col = lax.broadcasted_iota(jnp.int32, (_BLK_R, C), 1)
    drop = (col == lab_ref[...]) | (col >= C)
    e = jnp.where(drop, 0.0, jnp.exp(x))
    sum_ref[...] = jnp.sum(e, axis=1, keepdims=True)

    sub = lax.broadcasted_iota(jnp.int32, (8, 128), 0)
    groups = []
    for g in range(_G):
        acc = jnp.zeros((8, 128), jnp.float32)
        for j in range(8):
            acc = jnp.where(sub == j, tile_refs[8 * g + j][...], acc)
        groups.append(acc)
    stage_ref[...] = jnp.concatenate(groups, axis=0)[None]


def _make_tile_spec(k):
    def idx(rb, lab):
        r = rb * _BLK_R + k
        return (rb * _G + k // 8, lab[r] // 128)
    return pl.BlockSpec((8, 128), idx)


def _tc_stream(cos_theta, cos_theta_m, label, label2d, interpret=False):
    grid_spec = pltpu.PrefetchScalarGridSpec(
        num_scalar_prefetch=1,
        grid=(_RB,),
        in_specs=[
            pl.BlockSpec((_BLK_R, C), lambda rb, lab: (rb, 0)),
            *[_make_tile_spec(k) for k in range(_BLK_R)],
            pl.BlockSpec((_BLK_R, 1), lambda rb, lab: (rb, 0)),
        ],
        out_specs=[
            pl.BlockSpec((_BLK_R, 1), lambda rb, lab: (rb, 0)),
            pl.BlockSpec((1, _BLK_R, 128), lambda rb, lab: (rb, 0, 0)),
        ],
    )
    return pl.pallas_call(
        _tc_stream_body,
        grid_spec=grid_spec,
        out_shape=[
            jax.ShapeDtypeStruct((B, 1), jnp.float32),
            jax.ShapeDtypeStruct((_RB, _BLK_R, 128), jnp.float32),
        ],
        compiler_params=pltpu.CompilerParams(
            dimension_semantics=("arbitrary",),
        ),
        interpret=interpret,
    )(label, cos_theta, *([cos_theta_m] * _BLK_R), label2d)


_NC = 2
_NS = 16
_L = 16
_NW = _NC * _NS
_B_PER_W = B // _NW


def _sc_gather_kernel(staged_hbm, label_hbm, m_hbm, idx_v, val_v, sem):
    wid = lax.axis_index("s") * _NC + lax.axis_index("c")
    base = wid * _B_PER_W
    pltpu.sync_copy(label_hbm.at[pl.ds(base, _B_PER_W)], idx_v)
    for j in range(_B_PER_W // _L):
        lbl = idx_v[pl.ds(j * _L, _L)]
        rows = lax.iota(jnp.int32, _L) + (base + j * _L)
        idx_v[pl.ds(j * _L, _L)] = rows * 128 + (lbl & 127)
    pltpu.async_copy(staged_hbm.at[idx_v], val_v, sem).wait()
    pltpu.sync_copy(val_v, m_hbm.at[pl.ds(base, _B_PER_W)])


def _sc_gather(staged_flat, label):
    mesh = plsc.VectorSubcoreMesh(core_axis_name="c", subcore_axis_name="s")
    fn = functools.partial(
        pl.kernel,
        mesh=mesh,
        out_type=jax.ShapeDtypeStruct((B,), jnp.float32),
        scratch_types=[
            pltpu.VMEM((_B_PER_W,), jnp.int32),
            pltpu.VMEM((_B_PER_W,), jnp.float32),
            pltpu.SemaphoreType.DMA,
        ],
    )(_sc_gather_kernel)
    return fn(staged_flat, label)


def _tc_combine_body(sum_ref, m_ref, out_ref):
    sm = m_ref[...] * S
    total = sum_ref[...] + jnp.exp(sm)
    li = jnp.log(total) - sm
    out_ref[...] = jnp.sum(li, axis=0, keepdims=True) / B


def _tc_combine(sums, m2d, interpret=False):
    return pl.pallas_call(
        _tc_combine_body,
        out_shape=jax.ShapeDtypeStruct((1, 1), jnp.float32),
        interpret=interpret,
    )(sums, m2d)


def kernel(cos_theta, cos_theta_m, label):
    label = label.astype(jnp.int32)
    sums, staged = _tc_stream(cos_theta, cos_theta_m, label,
                              label.reshape(B, 1))
    m = _sc_gather(staged.reshape(B * 128), label)
    out = _tc_combine(sums, m.reshape(B, 1))
    return out[0, 0]
